# baseline (speedup 1.0000x reference)
import functools
import jax
import jax.numpy as jnp
from jax import lax
from jax.experimental import pallas as pl
from jax.experimental.pallas import tpu as pltpu
from jax.experimental.pallas import tpu_sc as plsc

N = 10000
D = 128
NH = N // 2
CH = 200
NCHJ = NH // CH
NBUF = 3


def _sc_body(nslab, x_hbm, emb_hbm, o_hbm, xbuf, ebuf, sx, se, so):
    spw = nslab // 16
    nk = NCHJ * spw
    c = lax.axis_index("c")
    g = lax.axis_index("s")
    n0h = c * NH

    def xcopy(k, slot):
        j, i = divmod(k, spw)
        slab = g * spw + i
        return pltpu.make_async_copy(
            x_hbm.at[slab, pl.ds(n0h + j * CH, CH)], xbuf.at[slot], sx.at[slot]
        )

    def ocopy(k, slot):
        j, i = divmod(k, spw)
        slab = g * spw + i
        return pltpu.make_async_copy(
            xbuf.at[slot], o_hbm.at[slab, pl.ds(n0h + j * CH, CH)], so.at[slot]
        )

    def ecopy(j, slot):
        return pltpu.make_async_copy(
            emb_hbm.at[pl.ds(n0h + j * CH, CH)], ebuf.at[slot], se.at[slot]
        )

    ecopy(0, 0).start()
    xcopy(0, 0).start()
    xcopy(1, 1).start()

    for k in range(nk):
        j, i = divmod(k, spw)
        kslot = k % NBUF
        jslot = j % 2
        if i == 0:
            ecopy(j, jslot).wait()
            if j + 1 < NCHJ:
                ecopy(j + 1, (j + 1) % 2).start()
        xcopy(k, kslot).wait()

        def row(r, _, kslot=kslot, jslot=jslot):
            for u in range(2):
                for kk in range(D // 16):
                    sl = pl.ds(kk * 16, 16)
                    plsc.addupdate(
                        xbuf.at[kslot, 2 * r + u, sl], ebuf[jslot, 2 * r + u, sl]
                    )
            return 0

        lax.fori_loop(0, CH // 2, row, 0)
        ocopy(k, kslot).start()
        if k + 2 < nk:
            if k >= 1:
                ocopy(k - 1, (k + 2) % NBUF).wait()
            xcopy(k + 2, (k + 2) % NBUF).start()

    for k in range(max(nk - 3, 0), nk):
        ocopy(k, k % NBUF).wait()


def sc_run(xt, emb_weight, nslab):
    mesh = plsc.VectorSubcoreMesh(core_axis_name="c", subcore_axis_name="s")
    run = pl.kernel(
        functools.partial(_sc_body, nslab),
        mesh=mesh,
        out_type=jax.ShapeDtypeStruct((nslab, N, D), xt.dtype),
        scratch_types=[
            pltpu.VMEM((NBUF, CH, D), xt.dtype),
            pltpu.VMEM((2, CH, D), xt.dtype),
            pltpu.SemaphoreType.DMA((NBUF,)),
            pltpu.SemaphoreType.DMA((2,)),
            pltpu.SemaphoreType.DMA((NBUF,)),
        ],
    )
    return run(xt, emb_weight)


K_SC = 16


def _tc_add_kernel(x_ref, emb_ref, o_ref):
    o_ref[...] = x_ref[...] + emb_ref[...][None, :, :]


def tc_run(xt, emb_weight):
    nslab, n, d = xt.shape
    return pl.pallas_call(
        _tc_add_kernel,
        grid=(1, nslab),
        in_specs=[
            pl.BlockSpec((1, n, d), lambda j, s: (s, j, 0)),
            pl.BlockSpec((n, d), lambda j, s: (j, 0)),
        ],
        out_specs=pl.BlockSpec((1, n, d), lambda j, s: (s, j, 0)),
        out_shape=jax.ShapeDtypeStruct((nslab, n, d), xt.dtype),
        compiler_params=pltpu.CompilerParams(
            dimension_semantics=("parallel", "parallel"),
        ),
    )(xt, emb_weight)


def kernel(x, emb_weight):
    batch, n, t, d = x.shape
    xt = jnp.transpose(x, (0, 2, 1, 3)).reshape(batch * t, n, d)
    sc_out = sc_run(xt[:K_SC], emb_weight, K_SC)
    tc_out = tc_run(xt[K_SC:], emb_weight)
    out = jnp.concatenate([sc_out, tc_out], axis=0)
    return jnp.transpose(out.reshape(batch, t, n, d), (0, 2, 1, 3))


# speedup vs baseline: 3.3449x; 3.3449x over previous
import jax
import jax.numpy as jnp
from jax.experimental import pallas as pl
from jax.experimental.pallas import tpu as pltpu

BN = 10000


def _add_kernel(x_ref, emb_ref, o_ref):
    o_ref[...] = x_ref[...] + emb_ref[...][None, :, :]


def kernel(x, emb_weight):
    batch, n, t, d = x.shape
    xt = jnp.transpose(x, (0, 2, 1, 3)).reshape(batch * t, n, d)
    out = pl.pallas_call(
        _add_kernel,
        grid=(n // BN, batch * t),
        in_specs=[
            pl.BlockSpec((1, BN, d), lambda j, s: (s, j, 0)),
            pl.BlockSpec((BN, d), lambda j, s: (j, 0)),
        ],
        out_specs=pl.BlockSpec((1, BN, d), lambda j, s: (s, j, 0)),
        out_shape=jax.ShapeDtypeStruct((batch * t, n, d), x.dtype),
        compiler_params=pltpu.CompilerParams(
            dimension_semantics=("parallel", "parallel"),
        ),
    )(xt, emb_weight)
    return jnp.transpose(out.reshape(batch, t, n, d), (0, 2, 1, 3))
